# baseline (speedup 1.0000x reference)
import jax
import jax.numpy as jnp
from jax.experimental import pallas as pl
from jax.experimental.pallas import tpu as pltpu

N_TOKENS = 16384
EMBED_DIM = 2048
LOW_RANK_DIM = 16
OUT_DIM = 64

CHUNK_T = 1024
N_BUF = 4
N_CHUNK = N_TOKENS // CHUNK_T


def _body(x_hbm, w1t_ref, w2t_ref, out_ref, xbuf, sems):
    def start(c):
        pltpu.make_async_copy(
            x_hbm.at[pl.ds(c * CHUNK_T, CHUNK_T), :],
            xbuf.at[c % N_BUF],
            sems.at[c % N_BUF],
        ).start()

    def wait(c):
        pltpu.make_async_copy(
            x_hbm.at[pl.ds(c * CHUNK_T, CHUNK_T), :],
            xbuf.at[c % N_BUF],
            sems.at[c % N_BUF],
        ).wait()

    for c in range(N_BUF):
        start(c)
    w1t = w1t_ref[...]
    w2t = w2t_ref[...]
    for c in range(N_CHUNK):
        wait(c)
        h = jnp.dot(xbuf[c % N_BUF], w1t, preferred_element_type=jnp.float32)
        out_ref[c * CHUNK_T:(c + 1) * CHUNK_T, :] = jnp.dot(
            h, w2t, preferred_element_type=jnp.float32)
        if c + N_BUF < N_CHUNK:
            start(c + N_BUF)


def kernel(x, w1, w2):
    n = x.shape[0]
    w1t = w1.T
    w2t = w2.T
    return pl.pallas_call(
        _body,
        in_specs=[
            pl.BlockSpec(memory_space=pl.ANY),
            pl.BlockSpec(memory_space=pltpu.MemorySpace.VMEM),
            pl.BlockSpec(memory_space=pltpu.MemorySpace.VMEM),
        ],
        out_specs=pl.BlockSpec(memory_space=pltpu.MemorySpace.VMEM),
        out_shape=jax.ShapeDtypeStruct((n, OUT_DIM), jnp.float32),
        scratch_shapes=[
            pltpu.VMEM((N_BUF, CHUNK_T, EMBED_DIM), jnp.float32),
            pltpu.SemaphoreType.DMA((N_BUF,)),
        ],
    )(x, w1t, w2t)
